# baseline (speedup 1.0000x reference)
import jax
import jax.numpy as jnp
from jax.experimental import pallas as pl
from jax.experimental.pallas import tpu as pltpu

_DIM = 2048
_HALF = _DIM // 2
_BS = 512


def _inv_freq(shape):
    d = jax.lax.broadcasted_iota(jnp.int32, shape, 1).astype(jnp.float32)
    return jnp.exp(d * (-jnp.log(10000.0) / _HALF))


def _make_kernel(nsb, seq):
    def _rope_kernel(labels_ref, x_ref, table_ref, o_ref, emb_ref):
        del labels_ref
        g = pl.program_id(0)

        @pl.when(g == 0)
        def _seed():
            k = jax.lax.broadcasted_iota(jnp.int32, (_BS, _HALF), 0).astype(
                jnp.float32
            )
            ang = k * _inv_freq((_BS, _HALF))
            emb_ref[:, :_HALF] = jnp.sin(ang)
            emb_ref[:, _HALF:] = jnp.cos(ang)

        @pl.when(g > 0)
        def _advance():
            delta = jnp.where(g % nsb == 0, jnp.float32(_BS - seq), jnp.float32(_BS))
            ang_d = delta * _inv_freq((1, _HALF))
            sin_d = jnp.sin(ang_d)
            cos_d = jnp.cos(ang_d)
            es = emb_ref[:, :_HALF]
            ec = emb_ref[:, _HALF:]
            emb_ref[:, :_HALF] = es * cos_d + ec * sin_d
            emb_ref[:, _HALF:] = ec * cos_d - es * sin_d

        le = table_ref[0, 0, :]
        o_ref[...] = x_ref[...] + emb_ref[...] * le[None, :]

    return _rope_kernel


def kernel(x, labels, label_table):
    batch, seq, dim = x.shape
    assert dim == _DIM and seq % _BS == 0
    nsb = seq // _BS
    labels = labels.astype(jnp.int32)
    x2 = x.reshape(batch * seq, dim)
    table3 = label_table.reshape(label_table.shape[0], 1, dim)
    out2 = pl.pallas_call(
        _make_kernel(nsb, seq),
        grid_spec=pltpu.PrefetchScalarGridSpec(
            num_scalar_prefetch=1,
            grid=(batch * nsb,),
            in_specs=[
                pl.BlockSpec((_BS, _DIM), lambda g, labels: (g, 0)),
                pl.BlockSpec((1, 1, _DIM), lambda g, labels: (labels[g // nsb], 0, 0)),
            ],
            out_specs=pl.BlockSpec((_BS, _DIM), lambda g, labels: (g, 0)),
            scratch_shapes=[
                pltpu.VMEM((_BS, _DIM), jnp.float32),
            ],
        ),
        out_shape=jax.ShapeDtypeStruct(x2.shape, x.dtype),
        compiler_params=pltpu.CompilerParams(
            dimension_semantics=("arbitrary",),
        ),
    )(labels, x2, table3)
    return out2.reshape(batch, seq, dim)
